# baseline (speedup 1.0000x reference)
import functools
import jax
import jax.numpy as jnp
from jax import lax
from jax.experimental import pallas as pl
from jax.experimental.pallas import tpu as pltpu, tpu_sc as plsc

_NC = 2
_NS = 16
_NW = _NC * _NS
_L = 16

_SEQ = 8192
_D = 1024
_B = 4

_CH_ROWS = 32
_CH = _CH_ROWS * _D
_ROWS_PER_W = _SEQ // _NW
_NCHUNK = _ROWS_PER_W // _CH_ROWS


def _sc_body(in_hbm, pos_hbm, out_hbm, p_v, a0, a1, ld0, ld1, st0, st1):
    wid = lax.axis_index("s") * _NC + lax.axis_index("c")
    seq_base = wid * _ROWS_PER_W * _D

    bufs = (a0, a1)
    ld_sems = (ld0, ld1)
    st_sems = (st0, st1)

    def add_chunk(buf):
        def body(i, _):
            sl = pl.ds(i * _L, _L)
            plsc.addupdate(buf.at[sl], p_v[sl])
            return 0

        lax.fori_loop(0, _CH // _L, body, 0)

    for c in range(_NCHUNK):
        pos_off = seq_base + c * _CH
        pltpu.sync_copy(pos_hbm.at[pl.ds(pos_off, _CH)], p_v)
        for b in range(_B):
            t = b % 2
            buf = bufs[t]
            off = b * _SEQ * _D + pos_off
            pltpu.async_copy(in_hbm.at[pl.ds(off, _CH)], buf, ld_sems[t]).wait()
            add_chunk(buf)
            pltpu.async_copy(buf, out_hbm.at[pl.ds(off, _CH)], st_sems[t]).wait()


def kernel(inputs, pos_table):
    batch, seq_len, out_dim = inputs.shape
    in_flat = inputs.reshape(batch * seq_len * out_dim)
    pos_flat = pos_table.reshape(seq_len * out_dim)
    mesh = plsc.VectorSubcoreMesh(core_axis_name="c", subcore_axis_name="s")
    out = pl.kernel(
        _sc_body,
        out_type=jax.ShapeDtypeStruct(in_flat.shape, in_flat.dtype),
        mesh=mesh,
        scratch_types=[
            pltpu.VMEM((_CH,), jnp.float32),
            pltpu.VMEM((_CH,), jnp.float32),
            pltpu.VMEM((_CH,), jnp.float32),
            pltpu.SemaphoreType.DMA,
            pltpu.SemaphoreType.DMA,
            pltpu.SemaphoreType.DMA,
            pltpu.SemaphoreType.DMA,
        ],
    )(in_flat, pos_flat)
    return out.reshape(batch, seq_len, out_dim)


# speedup vs baseline: 1.6375x; 1.6375x over previous
import jax
import jax.numpy as jnp
from jax import lax
from jax.experimental import pallas as pl
from jax.experimental.pallas import tpu as pltpu, tpu_sc as plsc

_NC = 2
_NS = 16
_NW = _NC * _NS
_L = 16

_SEQ = 8192
_D = 1024
_B = 4

_CH_ROWS = 16
_CH = _CH_ROWS * _D
_ROWS_PER_W = _SEQ // _NW
_NCHUNK = _ROWS_PER_W // _CH_ROWS
_NT = _NCHUNK * _B


def _sc_body(in_hbm, pos_hbm, out_hbm,
             p0, p1, a0, a1, pld0, pld1, ld0, ld1, st0, st1):
    wid = lax.axis_index("s") * _NC + lax.axis_index("c")
    seq_base = wid * _ROWS_PER_W * _D

    pbufs = (p0, p1)
    psems = (pld0, pld1)
    abufs = (a0, a1)
    lsems = (ld0, ld1)
    ssems = (st0, st1)

    def in_off(t):
        c, b = divmod(t, _B)
        return b * _SEQ * _D + seq_base + c * _CH

    def add_chunk(buf, pbuf):
        @plsc.parallel_loop(0, _CH // _L, unroll=8)
        def _(i):
            sl = pl.ds(i * _L, _L)
            plsc.addupdate(buf.at[sl], pbuf[sl])

    loads = [None] * _NT
    stores = [None] * _NT
    ploads = [None] * _NCHUNK

    ploads[0] = pltpu.async_copy(
        pos_hbm.at[pl.ds(seq_base, _CH)], p0, pld0)
    loads[0] = pltpu.async_copy(
        in_hbm.at[pl.ds(in_off(0), _CH)], a0, ld0)

    for t in range(_NT):
        c, b = divmod(t, _B)
        if b == 0 and c + 1 < _NCHUNK:
            ploads[c + 1] = pltpu.async_copy(
                pos_hbm.at[pl.ds(seq_base + (c + 1) * _CH, _CH)],
                pbufs[(c + 1) % 2], psems[(c + 1) % 2])
        if t + 1 < _NT:
            if t - 1 >= 0:
                stores[t - 1].wait()
            loads[t + 1] = pltpu.async_copy(
                in_hbm.at[pl.ds(in_off(t + 1), _CH)],
                abufs[(t + 1) % 2], lsems[(t + 1) % 2])
        loads[t].wait()
        if b == 0:
            ploads[c].wait()
        add_chunk(abufs[t % 2], pbufs[c % 2])
        stores[t] = pltpu.async_copy(
            abufs[t % 2], out_hbm.at[pl.ds(in_off(t), _CH)], ssems[t % 2])

    stores[_NT - 2].wait()
    stores[_NT - 1].wait()


def kernel(inputs, pos_table):
    batch, seq_len, out_dim = inputs.shape
    in_flat = inputs.reshape(batch * seq_len * out_dim)
    pos_flat = pos_table.reshape(seq_len * out_dim)
    mesh = plsc.VectorSubcoreMesh(core_axis_name="c", subcore_axis_name="s")
    out = pl.kernel(
        _sc_body,
        out_type=jax.ShapeDtypeStruct(in_flat.shape, in_flat.dtype),
        mesh=mesh,
        scratch_types=[
            pltpu.VMEM((_CH,), jnp.float32),
            pltpu.VMEM((_CH,), jnp.float32),
            pltpu.VMEM((_CH,), jnp.float32),
            pltpu.VMEM((_CH,), jnp.float32),
            pltpu.SemaphoreType.DMA,
            pltpu.SemaphoreType.DMA,
            pltpu.SemaphoreType.DMA,
            pltpu.SemaphoreType.DMA,
            pltpu.SemaphoreType.DMA,
            pltpu.SemaphoreType.DMA,
        ],
    )(in_flat, pos_flat)
    return out.reshape(batch, seq_len, out_dim)


# speedup vs baseline: 7.0453x; 4.3026x over previous
import jax
import jax.numpy as jnp
from jax.experimental import pallas as pl


_BLOCK_R = 1024


def kernel(inputs, pos_table):
    batch, seq_len, out_dim = inputs.shape
    ns = seq_len // _BLOCK_R
    flat = inputs.reshape(batch * seq_len, out_dim)

    def _add_body(x_ref, p_ref, o_ref):
        s = pl.program_id(0) % ns
        o_ref[...] = x_ref[...] + p_ref[pl.ds(s * _BLOCK_R, _BLOCK_R), :]

    out = pl.pallas_call(
        _add_body,
        grid=(batch * ns,),
        in_specs=[
            pl.BlockSpec((_BLOCK_R, out_dim), lambda i: (i, 0)),
            pl.BlockSpec(
                (seq_len, out_dim),
                lambda i: (0, 0),
                pipeline_mode=pl.Buffered(buffer_count=1),
            ),
        ],
        out_specs=pl.BlockSpec((_BLOCK_R, out_dim), lambda i: (i, 0)),
        out_shape=jax.ShapeDtypeStruct(flat.shape, flat.dtype),
    )(flat, pos_table)
    return out.reshape(batch, seq_len, out_dim)
